# baseline (speedup 1.0000x reference)
import functools

import jax
import jax.numpy as jnp
from jax import lax
from jax.experimental import pallas as pl
from jax.experimental.pallas import tpu as pltpu
from jax.experimental.pallas import tpu_sc as plsc

_IDX_CHUNK = 128
_BLK = 8192
_TBLK = 512


def _fold_body(embT_ref, featT_ref, w_ref, b_ref, o_ref):
    dim = w_ref.shape[1]
    eye = (lax.broadcasted_iota(jnp.int32, (dim, dim), 0)
           == lax.broadcasted_iota(jnp.int32, (dim, dim), 1)
           ).astype(jnp.float32)
    rhs = jnp.concatenate((eye, w_ref[...]), axis=0)
    dn = (((0,), (0,)), ((), ()))
    lhs_lo = jnp.concatenate(
        (embT_ref[:, :_BLK], featT_ref[:, :_BLK]), axis=0)
    lhs_hi = jnp.concatenate(
        (embT_ref[:, _BLK:], featT_ref[:, _BLK:]), axis=0)
    bias = b_ref[...]
    o_ref[:, :dim] = lax.dot_general(
        lhs_lo, rhs, dn, preferred_element_type=jnp.float32) + bias
    o_ref[:, dim:] = lax.dot_general(
        lhs_hi, rhs, dn, preferred_element_type=jnp.float32) + bias


@functools.lru_cache(maxsize=None)
def _build_fold(num_ids, dim, fdim):
    nblk = (num_ids + 2 * _BLK - 1) // (2 * _BLK)
    rows = nblk * _BLK
    return pl.pallas_call(
        _fold_body,
        grid=(nblk,),
        in_specs=[
            pl.BlockSpec((dim, 2 * _BLK), lambda i: (0, i)),
            pl.BlockSpec((fdim, 2 * _BLK), lambda i: (0, i)),
            pl.BlockSpec((fdim, dim), lambda i: (0, 0)),
            pl.BlockSpec((1, dim), lambda i: (0, 0)),
        ],
        out_specs=pl.BlockSpec((_BLK, 2 * dim), lambda i: (i, 0)),
        out_shape=jax.ShapeDtypeStruct((rows, 2 * dim), jnp.float32),
    )


@functools.lru_cache(maxsize=None)
def _build_gather(batch, rows2, dim):
    info = plsc.get_sparse_core_info()
    nw = info.num_cores * info.num_subcores
    nc = info.num_cores
    bpw = batch // nw
    ch = bpw // _IDX_CHUNK
    halfb = batch // 2

    mesh = plsc.VectorSubcoreMesh(core_axis_name="c", subcore_axis_name="s")

    @functools.partial(
        pl.kernel,
        mesh=mesh,
        compiler_params=pltpu.CompilerParams(use_tc_tiling_on_sc=False),
        out_type=jax.ShapeDtypeStruct((halfb, 2 * dim), jnp.float32),
        scratch_types=[
            pltpu.VMEM((ch, _IDX_CHUNK), jnp.int32),
            pltpu.VMEM((bpw, dim), jnp.float32),
            pltpu.SemaphoreType.DMA,
        ],
    )
    def gather(idx_hbm, tbl_hbm, out_hbm, idx_v, rows_v, sem):
        wid = lax.axis_index("s") * nc + lax.axis_index("c")
        pltpu.sync_copy(idx_hbm.at[pl.ds(wid * ch, ch)], idx_v)
        shift = _BLK.bit_length() - 1
        for r in range(ch):
            for k in range(_IDX_CHUNK // 16):
                v = idx_v[r, pl.ds(k * 16, 16)]
                vp = ((v & (-2 * _BLK))
                      + ((v & (_BLK - 1)) << 1)
                      + ((v >> shift) & 1))
                idx_v[r, pl.ds(k * 16, 16)] = vp
        copies = []
        for c in range(ch):
            copies.append(pltpu.async_copy(
                tbl_hbm.at[idx_v.at[c]],
                rows_v.at[pl.ds(c * _IDX_CHUNK, _IDX_CHUNK)], sem))
        for cp in copies:
            cp.wait()
        h = wid // (nw // 2)
        p0 = (wid % (nw // 2)) * bpw
        pltpu.sync_copy(rows_v,
                        out_hbm.at[pl.ds(p0, bpw), pl.ds(h * dim, dim)])

    return gather


@functools.lru_cache(maxsize=None)
def _build_tr(batch, dim):
    halfb = batch // 2
    nb = halfb // _TBLK

    def tr_body(g_ref, o_ref):
        h = pl.program_id(1)
        x = g_ref[...]
        xh = jnp.where(h == 0, x[:, :dim], x[:, dim:])
        o_ref[...] = xh.T

    return pl.pallas_call(
        tr_body,
        grid=(nb, 2),
        in_specs=[pl.BlockSpec((_TBLK, 2 * dim), lambda j, h: (j, 0))],
        out_specs=pl.BlockSpec((dim, _TBLK), lambda j, h: (0, h * nb + j)),
        out_shape=jax.ShapeDtypeStruct((dim, batch), jnp.float32),
    )


@jax.jit
def kernel(inputs, emb_table, feature_table, W, b):
    batch = inputs.shape[0]
    num_ids, dim = emb_table.shape
    fdim = feature_table.shape[1]

    packed = _build_fold(num_ids, dim, fdim)(
        emb_table.T, feature_table.T, W, b.reshape(1, dim))
    tbl = packed.reshape(2 * packed.shape[0], dim)

    idx = inputs.astype(jnp.int32).reshape(batch // _IDX_CHUNK, _IDX_CHUNK)
    g = _build_gather(batch, tbl.shape[0], dim)(idx, tbl)
    return _build_tr(batch, dim)(g).T


# speedup vs baseline: 1.1134x; 1.1134x over previous
import functools

import jax
import jax.numpy as jnp
from jax import lax
from jax.experimental import pallas as pl
from jax.experimental.pallas import tpu as pltpu
from jax.experimental.pallas import tpu_sc as plsc

_IDX_CHUNK = 128
_BLK = 8192
_TBLK = 1024


def _fold_body(embT_ref, featT_ref, w_ref, b_ref, o_ref):
    dim = w_ref.shape[1]
    eye = (lax.broadcasted_iota(jnp.int32, (dim, dim), 0)
           == lax.broadcasted_iota(jnp.int32, (dim, dim), 1)
           ).astype(jnp.float32)
    rhs = jnp.concatenate((eye, w_ref[...]), axis=0)
    dn = (((0,), (0,)), ((), ()))
    lhs_lo = jnp.concatenate(
        (embT_ref[:, :_BLK], featT_ref[:, :_BLK]), axis=0)
    lhs_hi = jnp.concatenate(
        (embT_ref[:, _BLK:], featT_ref[:, _BLK:]), axis=0)
    bias = b_ref[...]
    o_ref[:, :dim] = lax.dot_general(
        lhs_lo, rhs, dn, preferred_element_type=jnp.float32) + bias
    o_ref[:, dim:] = lax.dot_general(
        lhs_hi, rhs, dn, preferred_element_type=jnp.float32) + bias


@functools.lru_cache(maxsize=None)
def _build_fold(num_ids, dim, fdim):
    nblk = (num_ids + 2 * _BLK - 1) // (2 * _BLK)
    rows = nblk * _BLK
    return pl.pallas_call(
        _fold_body,
        grid=(nblk,),
        in_specs=[
            pl.BlockSpec((dim, 2 * _BLK), lambda i: (0, i)),
            pl.BlockSpec((fdim, 2 * _BLK), lambda i: (0, i)),
            pl.BlockSpec((fdim, dim), lambda i: (0, 0)),
            pl.BlockSpec((1, dim), lambda i: (0, 0)),
        ],
        out_specs=pl.BlockSpec((_BLK, 2 * dim), lambda i: (i, 0)),
        out_shape=jax.ShapeDtypeStruct((rows, 2 * dim), jnp.float32),
    )


@functools.lru_cache(maxsize=None)
def _build_gather(batch, rows2, dim):
    info = plsc.get_sparse_core_info()
    nw = info.num_cores * info.num_subcores
    nc = info.num_cores
    bpw = batch // nw
    ch = bpw // _IDX_CHUNK
    halfb = batch // 2

    mesh = plsc.VectorSubcoreMesh(core_axis_name="c", subcore_axis_name="s")

    @functools.partial(
        pl.kernel,
        mesh=mesh,
        compiler_params=pltpu.CompilerParams(use_tc_tiling_on_sc=False),
        out_type=jax.ShapeDtypeStruct((halfb, 2 * dim), jnp.float32),
        scratch_types=[
            pltpu.VMEM((ch, _IDX_CHUNK), jnp.int32),
            pltpu.VMEM((bpw, dim), jnp.float32),
            pltpu.SemaphoreType.DMA,
        ],
    )
    def gather(idx_hbm, tbl_hbm, out_hbm, idx_v, rows_v, sem):
        wid = lax.axis_index("s") * nc + lax.axis_index("c")
        pltpu.sync_copy(idx_hbm.at[pl.ds(wid * ch, ch)], idx_v)
        shift = _BLK.bit_length() - 1
        for r in range(ch):
            for k in range(_IDX_CHUNK // 16):
                v = idx_v[r, pl.ds(k * 16, 16)]
                vp = ((v & (-2 * _BLK))
                      + ((v & (_BLK - 1)) << 1)
                      + ((v >> shift) & 1))
                idx_v[r, pl.ds(k * 16, 16)] = vp
        copies = []
        for c in range(ch):
            copies.append(pltpu.async_copy(
                tbl_hbm.at[idx_v.at[c]],
                rows_v.at[pl.ds(c * _IDX_CHUNK, _IDX_CHUNK)], sem))
        for cp in copies:
            cp.wait()
        h = wid // (nw // 2)
        p0 = (wid % (nw // 2)) * bpw
        pltpu.sync_copy(rows_v,
                        out_hbm.at[pl.ds(p0, bpw), pl.ds(h * dim, dim)])

    return gather


@functools.lru_cache(maxsize=None)
def _build_tr(batch, dim):
    halfb = batch // 2
    nb = halfb // _TBLK

    def tr_body(g_ref, o_ref):
        h = pl.program_id(1)
        x = g_ref[...]
        xh = jnp.where(h == 0, x[:, :dim], x[:, dim:])
        o_ref[...] = xh.T

    return pl.pallas_call(
        tr_body,
        grid=(nb, 2),
        in_specs=[pl.BlockSpec((_TBLK, 2 * dim), lambda j, h: (j, 0))],
        out_specs=pl.BlockSpec((dim, _TBLK), lambda j, h: (0, h * nb + j)),
        out_shape=jax.ShapeDtypeStruct((dim, batch), jnp.float32),
    )


@jax.jit
def kernel(inputs, emb_table, feature_table, W, b):
    batch = inputs.shape[0]
    num_ids, dim = emb_table.shape
    fdim = feature_table.shape[1]

    packed = _build_fold(num_ids, dim, fdim)(
        emb_table.T, feature_table.T, W, b.reshape(1, dim))
    tbl = packed.reshape(2 * packed.shape[0], dim)

    idx = inputs.astype(jnp.int32).reshape(batch // _IDX_CHUNK, _IDX_CHUNK)
    g = _build_gather(batch, tbl.shape[0], dim)(idx, tbl)
    return _build_tr(batch, dim)(g).T


# speedup vs baseline: 1.2149x; 1.0912x over previous
import functools

import jax
import jax.numpy as jnp
from jax import lax
from jax.experimental import pallas as pl
from jax.experimental.pallas import tpu as pltpu
from jax.experimental.pallas import tpu_sc as plsc

_IDX_CHUNK = 128
_BLK = 8192
_TBLK = 4096


def _fold_body(embT_ref, featT_ref, w_ref, b_ref, o_ref):
    dim = w_ref.shape[1]
    eye = (lax.broadcasted_iota(jnp.int32, (dim, dim), 0)
           == lax.broadcasted_iota(jnp.int32, (dim, dim), 1)
           ).astype(jnp.float32)
    rhs = jnp.concatenate((eye, w_ref[...]), axis=0)
    dn = (((0,), (0,)), ((), ()))
    lhs_lo = jnp.concatenate(
        (embT_ref[:, :_BLK], featT_ref[:, :_BLK]), axis=0)
    lhs_hi = jnp.concatenate(
        (embT_ref[:, _BLK:], featT_ref[:, _BLK:]), axis=0)
    bias = b_ref[...]
    o_ref[:, :dim] = lax.dot_general(
        lhs_lo, rhs, dn, preferred_element_type=jnp.float32) + bias
    o_ref[:, dim:] = lax.dot_general(
        lhs_hi, rhs, dn, preferred_element_type=jnp.float32) + bias


@functools.lru_cache(maxsize=None)
def _build_fold(num_ids, dim, fdim):
    nblk = (num_ids + 2 * _BLK - 1) // (2 * _BLK)
    rows = nblk * _BLK
    return pl.pallas_call(
        _fold_body,
        grid=(nblk,),
        in_specs=[
            pl.BlockSpec((dim, 2 * _BLK), lambda i: (0, i)),
            pl.BlockSpec((fdim, 2 * _BLK), lambda i: (0, i)),
            pl.BlockSpec((fdim, dim), lambda i: (0, 0)),
            pl.BlockSpec((1, dim), lambda i: (0, 0)),
        ],
        out_specs=pl.BlockSpec((_BLK, 2 * dim), lambda i: (i, 0)),
        out_shape=jax.ShapeDtypeStruct((rows, 2 * dim), jnp.float32),
    )


@functools.lru_cache(maxsize=None)
def _build_gather(batch, rows2, dim):
    info = plsc.get_sparse_core_info()
    nw = info.num_cores * info.num_subcores
    nc = info.num_cores
    bpw = batch // nw
    ch = bpw // _IDX_CHUNK
    halfb = batch // 2

    mesh = plsc.VectorSubcoreMesh(core_axis_name="c", subcore_axis_name="s")

    @functools.partial(
        pl.kernel,
        mesh=mesh,
        compiler_params=pltpu.CompilerParams(use_tc_tiling_on_sc=False),
        out_type=jax.ShapeDtypeStruct((halfb, 2 * dim), jnp.float32),
        scratch_types=[
            pltpu.VMEM((ch, _IDX_CHUNK), jnp.int32),
            pltpu.VMEM((bpw, dim), jnp.float32),
            pltpu.SemaphoreType.DMA,
        ],
    )
    def gather(idx_hbm, tbl_hbm, out_hbm, idx_v, rows_v, sem):
        wid = lax.axis_index("s") * nc + lax.axis_index("c")
        pltpu.sync_copy(idx_hbm.at[pl.ds(wid * ch, ch)], idx_v)
        shift = _BLK.bit_length() - 1
        for r in range(ch):
            for k in range(_IDX_CHUNK // 16):
                v = idx_v[r, pl.ds(k * 16, 16)]
                vp = ((v & (-2 * _BLK))
                      + ((v & (_BLK - 1)) << 1)
                      + ((v >> shift) & 1))
                idx_v[r, pl.ds(k * 16, 16)] = vp
        copies = []
        for c in range(ch):
            copies.append(pltpu.async_copy(
                tbl_hbm.at[idx_v.at[c]],
                rows_v.at[pl.ds(c * _IDX_CHUNK, _IDX_CHUNK)], sem))
        for cp in copies:
            cp.wait()
        h = wid // (nw // 2)
        p0 = (wid % (nw // 2)) * bpw
        pltpu.sync_copy(rows_v,
                        out_hbm.at[pl.ds(p0, bpw), pl.ds(h * dim, dim)])

    return gather


@functools.lru_cache(maxsize=None)
def _build_tr(batch, dim):
    halfb = batch // 2
    nb = halfb // _TBLK

    def tr_body(g_ref, o_ref):
        h = pl.program_id(1)
        x = g_ref[...]
        xh = jnp.where(h == 0, x[:, :dim], x[:, dim:])
        o_ref[...] = xh.T

    return pl.pallas_call(
        tr_body,
        grid=(nb, 2),
        in_specs=[pl.BlockSpec((_TBLK, 2 * dim), lambda j, h: (j, 0))],
        out_specs=pl.BlockSpec((dim, _TBLK), lambda j, h: (0, h * nb + j)),
        out_shape=jax.ShapeDtypeStruct((dim, batch), jnp.float32),
    )


@jax.jit
def kernel(inputs, emb_table, feature_table, W, b):
    batch = inputs.shape[0]
    num_ids, dim = emb_table.shape
    fdim = feature_table.shape[1]

    packed = _build_fold(num_ids, dim, fdim)(
        emb_table.T, feature_table.T, W, b.reshape(1, dim))
    tbl = packed.reshape(2 * packed.shape[0], dim)

    idx = inputs.astype(jnp.int32).reshape(batch // _IDX_CHUNK, _IDX_CHUNK)
    g = _build_gather(batch, tbl.shape[0], dim)(idx, tbl)
    return _build_tr(batch, dim)(g).T


# speedup vs baseline: 1.2189x; 1.0032x over previous
import functools

import jax
import jax.numpy as jnp
from jax import lax
from jax.experimental import pallas as pl
from jax.experimental.pallas import tpu as pltpu
from jax.experimental.pallas import tpu_sc as plsc

_IDX_CHUNK = 128
_BLK = 8192
_TBLK = 8192


def _fold_body(embT_ref, featT_ref, w_ref, b_ref, o_ref):
    dim = w_ref.shape[1]
    eye = (lax.broadcasted_iota(jnp.int32, (dim, dim), 0)
           == lax.broadcasted_iota(jnp.int32, (dim, dim), 1)
           ).astype(jnp.float32)
    rhs = jnp.concatenate((eye, w_ref[...]), axis=0)
    dn = (((0,), (0,)), ((), ()))
    lhs_lo = jnp.concatenate(
        (embT_ref[:, :_BLK], featT_ref[:, :_BLK]), axis=0)
    lhs_hi = jnp.concatenate(
        (embT_ref[:, _BLK:], featT_ref[:, _BLK:]), axis=0)
    bias = b_ref[...]
    o_ref[:, :dim] = lax.dot_general(
        lhs_lo, rhs, dn, preferred_element_type=jnp.float32) + bias
    o_ref[:, dim:] = lax.dot_general(
        lhs_hi, rhs, dn, preferred_element_type=jnp.float32) + bias


@functools.lru_cache(maxsize=None)
def _build_fold(num_ids, dim, fdim):
    nblk = (num_ids + 2 * _BLK - 1) // (2 * _BLK)
    rows = nblk * _BLK
    return pl.pallas_call(
        _fold_body,
        grid=(nblk,),
        in_specs=[
            pl.BlockSpec((dim, 2 * _BLK), lambda i: (0, i)),
            pl.BlockSpec((fdim, 2 * _BLK), lambda i: (0, i)),
            pl.BlockSpec((fdim, dim), lambda i: (0, 0)),
            pl.BlockSpec((1, dim), lambda i: (0, 0)),
        ],
        out_specs=pl.BlockSpec((_BLK, 2 * dim), lambda i: (i, 0)),
        out_shape=jax.ShapeDtypeStruct((rows, 2 * dim), jnp.float32),
    )


@functools.lru_cache(maxsize=None)
def _build_gather(batch, rows2, dim):
    info = plsc.get_sparse_core_info()
    nw = info.num_cores * info.num_subcores
    nc = info.num_cores
    bpw = batch // nw
    ch = bpw // _IDX_CHUNK
    halfb = batch // 2

    mesh = plsc.VectorSubcoreMesh(core_axis_name="c", subcore_axis_name="s")

    @functools.partial(
        pl.kernel,
        mesh=mesh,
        compiler_params=pltpu.CompilerParams(use_tc_tiling_on_sc=False),
        out_type=jax.ShapeDtypeStruct((halfb, 2 * dim), jnp.float32),
        scratch_types=[
            pltpu.VMEM((ch, _IDX_CHUNK), jnp.int32),
            pltpu.VMEM((bpw, dim), jnp.float32),
            pltpu.SemaphoreType.DMA,
        ],
    )
    def gather(idx_hbm, tbl_hbm, out_hbm, idx_v, rows_v, sem):
        wid = lax.axis_index("s") * nc + lax.axis_index("c")
        pltpu.sync_copy(idx_hbm.at[pl.ds(wid * ch, ch)], idx_v)
        shift = _BLK.bit_length() - 1
        for r in range(ch):
            for k in range(_IDX_CHUNK // 16):
                v = idx_v[r, pl.ds(k * 16, 16)]
                vp = ((v & (-2 * _BLK))
                      + ((v & (_BLK - 1)) << 1)
                      + ((v >> shift) & 1))
                idx_v[r, pl.ds(k * 16, 16)] = vp
        copies = []
        for c in range(ch):
            copies.append(pltpu.async_copy(
                tbl_hbm.at[idx_v.at[c]],
                rows_v.at[pl.ds(c * _IDX_CHUNK, _IDX_CHUNK)], sem))
        for cp in copies:
            cp.wait()
        h = wid // (nw // 2)
        p0 = (wid % (nw // 2)) * bpw
        pltpu.sync_copy(rows_v,
                        out_hbm.at[pl.ds(p0, bpw), pl.ds(h * dim, dim)])

    return gather


@functools.lru_cache(maxsize=None)
def _build_tr(batch, dim):
    halfb = batch // 2
    nb = halfb // _TBLK

    def tr_body(g_ref, o_ref):
        h = pl.program_id(1)
        x = g_ref[...]
        xh = jnp.where(h == 0, x[:, :dim], x[:, dim:])
        o_ref[...] = xh.T

    return pl.pallas_call(
        tr_body,
        grid=(nb, 2),
        in_specs=[pl.BlockSpec((_TBLK, 2 * dim), lambda j, h: (j, 0))],
        out_specs=pl.BlockSpec((dim, _TBLK), lambda j, h: (0, h * nb + j)),
        out_shape=jax.ShapeDtypeStruct((dim, batch), jnp.float32),
    )


@jax.jit
def kernel(inputs, emb_table, feature_table, W, b):
    batch = inputs.shape[0]
    num_ids, dim = emb_table.shape
    fdim = feature_table.shape[1]

    packed = _build_fold(num_ids, dim, fdim)(
        emb_table.T, feature_table.T, W, b.reshape(1, dim))
    tbl = packed.reshape(2 * packed.shape[0], dim)

    idx = inputs.astype(jnp.int32).reshape(batch // _IDX_CHUNK, _IDX_CHUNK)
    g = _build_gather(batch, tbl.shape[0], dim)(idx, tbl)
    return _build_tr(batch, dim)(g).T
